# Initial kernel scaffold; baseline (speedup 1.0000x reference)
import jax
import jax.numpy as jnp
from jax.experimental import pallas as pl


def kernel(x, edge_index, W0, b0, W1, b1, Wc):
    raise NotImplementedError("write your pallas kernel here")


# speedup vs baseline: 5.4390x; 5.4390x over previous
import functools

import numpy as np
import jax
import jax.numpy as jnp
from jax import lax
from jax.experimental import pallas as pl
from jax.experimental.pallas import tpu as pltpu
from jax.experimental.pallas import tpu_sc as plsc

_N = 10000
_E = 320000
_HID = 128
_OUT = 64
_LAYERS = 4
_ALPHA = 0.1
_THETA = 0.5

_CHUNK = 128
_NCHUNKS = _E // _CHUNK
_NCORES = 2
_NSUB = 16
_NW = _NCORES * _NSUB
_NZ = 40
_ZCHUNKS = _N // _NZ

_ROWBLK = 1000
_GRID = _N // _ROWBLK


def _sc_aggregate(h, src2d, dst2d):
    mesh = plsc.VectorSubcoreMesh(core_axis_name="c", subcore_axis_name="s")

    @functools.partial(
        pl.kernel,
        out_type=jax.ShapeDtypeStruct((_NCORES, _N, _HID), jnp.float32),
        mesh=mesh,
        scratch_types=[
            pltpu.VMEM((1, _CHUNK), jnp.int32),
            pltpu.VMEM((1, _CHUNK), jnp.int32),
            pltpu.VMEM((_CHUNK, _HID), jnp.float32),
            pltpu.VMEM((_NZ, _HID), jnp.float32),
            pltpu.VMEM_SHARED((_N, _HID), jnp.float32),
            pltpu.SemaphoreType.DMA,
        ],
    )
    def k(h_hbm, src_hbm, dst_hbm, out_hbm, sidx, didx, rows, zbuf, agg, sem):
        cid = lax.axis_index("c")
        sid = lax.axis_index("s")
        wid = cid * _NSUB + sid

        zero = jnp.zeros((16,), jnp.float32)

        @pl.loop(0, _NZ)
        def _(r):
            for c0 in range(0, _HID, 16):
                zbuf[r, pl.ds(c0, 16)] = zero

        @pl.loop(sid, _ZCHUNKS, step=_NSUB)
        def _(z):
            pltpu.sync_copy(zbuf, agg.at[pl.ds(z * _NZ, _NZ)])

        plsc.subcore_barrier()

        @pl.loop(wid, _NCHUNKS, step=_NW)
        def _(j):
            pltpu.sync_copy(src_hbm.at[pl.ds(j, 1)], sidx)
            pltpu.sync_copy(dst_hbm.at[pl.ds(j, 1)], didx)
            pltpu.async_copy(h_hbm.at[sidx.at[0]], rows, sem).wait()
            pltpu.sync_copy(rows, agg.at[didx.at[0]], add=True)

        plsc.subcore_barrier()

        @pl.loop(sid, _ZCHUNKS, step=_NSUB)
        def _(z):
            pltpu.sync_copy(agg.at[pl.ds(z * _NZ, _NZ)],
                            out_hbm.at[cid, pl.ds(z * _NZ, _NZ)])

    return k(h, src2d, dst2d)


def _tc_entry(x, w0t, b0):
    def body(x_ref, w_ref, b_ref, o_ref):
        y = jnp.dot(x_ref[...], w_ref[...], preferred_element_type=jnp.float32)
        o_ref[...] = jnp.maximum(y + b_ref[...], 0.0)

    return pl.pallas_call(
        body,
        grid=(_GRID,),
        in_specs=[
            pl.BlockSpec((_ROWBLK, _HID), lambda i: (i, 0)),
            pl.BlockSpec((_HID, _HID), lambda i: (0, 0)),
            pl.BlockSpec((1, _HID), lambda i: (0, 0)),
        ],
        out_specs=pl.BlockSpec((_ROWBLK, _HID), lambda i: (i, 0)),
        out_shape=jax.ShapeDtypeStruct((_N, _HID), jnp.float32),
    )(x, w0t, b0)


def _tc_layer(parts, x0, wc_l, beta):
    one_m_a = 1.0 - _ALPHA
    one_m_b = 1.0 - beta

    def body(pa_ref, pb_ref, x0_ref, w_ref, o_ref):
        agg = pa_ref[0] + pb_ref[0]
        hh = one_m_a * agg + _ALPHA * x0_ref[...]
        y = jnp.dot(hh, w_ref[...], preferred_element_type=jnp.float32)
        o_ref[...] = jnp.maximum(one_m_b * hh + beta * y, 0.0)

    return pl.pallas_call(
        body,
        grid=(_GRID,),
        in_specs=[
            pl.BlockSpec((1, _ROWBLK, _HID), lambda i: (0, i, 0)),
            pl.BlockSpec((1, _ROWBLK, _HID), lambda i: (1, i, 0)),
            pl.BlockSpec((_ROWBLK, _HID), lambda i: (i, 0)),
            pl.BlockSpec((_HID, _HID), lambda i: (0, 0)),
        ],
        out_specs=pl.BlockSpec((_ROWBLK, _HID), lambda i: (i, 0)),
        out_shape=jax.ShapeDtypeStruct((_N, _HID), jnp.float32),
    )(parts, parts, x0, wc_l)


def _tc_final(h, w1t, b1):
    def body(h_ref, w_ref, b_ref, o_ref):
        y = jnp.dot(h_ref[...], w_ref[...], preferred_element_type=jnp.float32)
        y = y + b_ref[...]
        m = jnp.max(y, axis=-1, keepdims=True)
        e = jnp.exp(y - m)
        lse = jnp.log(jnp.sum(e, axis=-1, keepdims=True))
        o_ref[...] = y - m - lse

    return pl.pallas_call(
        body,
        grid=(_GRID,),
        in_specs=[
            pl.BlockSpec((_ROWBLK, _HID), lambda i: (i, 0)),
            pl.BlockSpec((_HID, _OUT), lambda i: (0, 0)),
            pl.BlockSpec((1, _OUT), lambda i: (0, 0)),
        ],
        out_specs=pl.BlockSpec((_ROWBLK, _OUT), lambda i: (i, 0)),
        out_shape=jax.ShapeDtypeStruct((_N, _OUT), jnp.float32),
    )(h, w1t, b1)


def kernel(x, edge_index, W0, b0, W1, b1, Wc):
    ei = edge_index.astype(jnp.int32)
    src2d = ei[0].reshape(_NCHUNKS, _CHUNK)
    dst2d = ei[1].reshape(_NCHUNKS, _CHUNK)

    h = _tc_entry(x, W0.T, b0.reshape(1, _HID))
    x0 = h
    for l in range(_LAYERS):
        parts = _sc_aggregate(h, src2d, dst2d)
        beta = float(np.log(_THETA / (l + 1) + 1.0))
        h = _tc_layer(parts, x0, Wc[l], beta)
    return _tc_final(h, W1.T, b1.reshape(1, _OUT))
